# baseline (speedup 1.0000x reference)
import functools

import jax
import jax.numpy as jnp
from jax import lax
from jax.experimental import pallas as pl
from jax.experimental.pallas import tpu as pltpu, tpu_sc as plsc

_TOP_K = 8
_E = 64
_T = 2048


def _matmul_kernel(hs_ref, w_ref, out_ref):
    out_ref[...] = jax.lax.dot_general(
        hs_ref[...], w_ref[...], (((1,), (1,)), ((), ())),
        preferred_element_type=jnp.float32,
    )


def _tc_logits(hs, weight):
    n, h = hs.shape
    e = weight.shape[0]
    return pl.pallas_call(
        _matmul_kernel,
        grid=(n // _T,),
        in_specs=[
            pl.BlockSpec((_T, h), lambda i: (i, 0)),
            pl.BlockSpec((e, h), lambda i: (0, 0)),
        ],
        out_specs=pl.BlockSpec((_T, e), lambda i: (i, 0)),
        out_shape=jax.ShapeDtypeStruct((n, e), jnp.float32),
    )(hs, weight)


def _sc_topk(logits):
    n = logits.shape[0]
    nc, ns = 2, 16
    nw = nc * ns
    t_per_w = n // nw
    mesh = plsc.VectorSubcoreMesh(core_axis_name="c", subcore_axis_name="s")

    @functools.partial(
        pl.kernel,
        out_type=[
            jax.ShapeDtypeStruct((n, 16), jnp.int32),
            jax.ShapeDtypeStruct((n, 16), jnp.float32),
        ],
        mesh=mesh,
        scratch_types=[
            pltpu.VMEM((t_per_w, _E), jnp.float32),
            pltpu.VMEM((32,), jnp.float32),
            pltpu.VMEM((32,), jnp.int32),
            pltpu.VMEM((t_per_w, 16), jnp.int32),
            pltpu.VMEM((t_per_w, 16), jnp.float32),
        ],
        compiler_params=pltpu.CompilerParams(needs_layout_passes=False),
    )
    def k(logits_hbm, idx_hbm, wgt_hbm, slab, stk, stv, oidx, owgt):
        wid = lax.axis_index("s") * nc + lax.axis_index("c")
        base = wid * t_per_w
        pltpu.sync_copy(logits_hbm.at[pl.ds(base, t_per_w)], slab)
        lane = lax.iota(jnp.int32, 16)
        lane_f = lane.astype(jnp.float32)

        def merge(ak, av, bk, bv):
            stk[pl.ds(0, 16)] = ak
            stv[pl.ds(0, 16)] = av
            stk[pl.ds(8, 16)] = bk
            stv[pl.ds(8, 16)] = bv
            ck = stk[pl.ds(0, 16)]
            cv = stv[pl.ds(0, 16)]
            return plsc.sort_key_val(ck, cv, descending=True)

        def body(t, carry):
            s = []
            for j in range(4):
                kj = slab[t, pl.ds(16 * j, 16)]
                ij = lane + 16 * j
                s.append(plsc.sort_key_val(kj, ij, descending=True))
            t01k, t01v = merge(s[0][0], s[0][1], s[1][0], s[1][1])
            t23k, t23v = merge(s[2][0], s[2][1], s[3][0], s[3][1])
            fk, fv = merge(t01k, t01v, t23k, t23v)
            mx = lax.reduce_max(fk, (0,))
            ex = jnp.exp(fk - mx)
            ssum = lax.reduce_sum(jnp.where(lane_f < 8.0, ex, 0.0), (0,))
            owgt[t, :] = ex / ssum
            oidx[t, :] = fv
            return carry

        lax.fori_loop(0, t_per_w, body, 0)
        pltpu.sync_copy(oidx, idx_hbm.at[pl.ds(base, t_per_w)])
        pltpu.sync_copy(owgt, wgt_hbm.at[pl.ds(base, t_per_w)])

    return k(logits)


def kernel(hidden_states, weight):
    b, s, h = hidden_states.shape
    hs = hidden_states.reshape(-1, h)
    logits = _tc_logits(hs, weight)
    idx16, wgt16 = _sc_topk(logits)
    return idx16[:, :_TOP_K], wgt16[:, :_TOP_K]


# speedup vs baseline: 2.0390x; 2.0390x over previous
import jax
import jax.numpy as jnp
from jax.experimental import pallas as pl

_TOP_K = 8
_T = 2048
_C = 512


def _gate_kernel(hs_ref, w_ref, idx_ref, wgt_ref):
    logits_t = jax.lax.dot_general(
        w_ref[...], hs_ref[...], (((1,), (1,)), ((), ())),
        preferred_element_type=jnp.float32,
    )
    e = logits_t.shape[0]
    iota_f = jax.lax.broadcasted_iota(jnp.int32, (e, _C), 0).astype(jnp.float32)
    kiota_f = jax.lax.broadcasted_iota(jnp.int32, (_TOP_K, _C), 0).astype(
        jnp.float32
    )
    for c in range(_T // _C):
        cur = logits_t[:, c * _C : (c + 1) * _C]
        vtop = jnp.zeros((_TOP_K, _C), jnp.float32)
        itop = jnp.zeros((_TOP_K, _C), jnp.float32)
        for k in range(_TOP_K):
            m = jnp.max(cur, axis=0, keepdims=True)
            is_max = cur == m
            i = jnp.min(
                jnp.where(is_max, iota_f, float(e)), axis=0, keepdims=True
            )
            vtop = jnp.where(kiota_f == float(k), m, vtop)
            itop = jnp.where(kiota_f == float(k), i, itop)
            cur = jnp.where(iota_f == i, -jnp.inf, cur)
        ex = jnp.exp(vtop - vtop[:1, :])
        wgt = ex / jnp.sum(ex, axis=0, keepdims=True)
        idx_ref[c * _C : (c + 1) * _C, :] = itop.astype(jnp.int32).T
        wgt_ref[c * _C : (c + 1) * _C, :] = wgt.T


def kernel(hidden_states, weight):
    b, s, h = hidden_states.shape
    hs = hidden_states.reshape(-1, h)
    n = hs.shape[0]
    e = weight.shape[0]
    grid = n // _T
    idx, wgt = pl.pallas_call(
        _gate_kernel,
        grid=(grid,),
        in_specs=[
            pl.BlockSpec((_T, h), lambda i: (i, 0)),
            pl.BlockSpec((e, h), lambda i: (0, 0)),
        ],
        out_specs=[
            pl.BlockSpec((_T, _TOP_K), lambda i: (i, 0)),
            pl.BlockSpec((_T, _TOP_K), lambda i: (i, 0)),
        ],
        out_shape=[
            jax.ShapeDtypeStruct((n, _TOP_K), jnp.int32),
            jax.ShapeDtypeStruct((n, _TOP_K), jnp.float32),
        ],
    )(hs, weight)
    return idx, wgt
